# baseline (speedup 1.0000x reference)
import functools

import jax
import jax.numpy as jnp
from jax import lax
from jax.experimental import pallas as pl
from jax.experimental.pallas import tpu as pltpu
from jax.experimental.pallas import tpu_sc as plsc

N_ROWS = 9216
N_EMB = 1024
DIM = 64
BN = 1024
N_BLOCKS = N_ROWS // BN


def _dist_argmin_body(x_ref, emb_ref, xsq_ref, esq_ref, idx_ref, loss_ref):
    x = x_ref[...]
    emb = emb_ref[...]
    x_sq = xsq_ref[...]
    e_sq = esq_ref[...]
    cross = lax.dot_general(
        x, emb, (((1,), (1,)), ((), ())),
        preferred_element_type=jnp.float32)
    dis2 = jnp.maximum(x_sq + e_sq - 2.0 * cross, 0.0)
    dis = jnp.sqrt(dis2)
    mn = jnp.min(dis, axis=1, keepdims=True)
    iota = lax.broadcasted_iota(jnp.int32, dis.shape, 1)
    idx = jnp.min(jnp.where(dis == mn, iota, N_EMB), axis=1)
    idx_ref[0, 0, :] = idx
    blk = jnp.sum(jnp.min(dis2, axis=1))
    prev = jnp.where(pl.program_id(0) == 0,
                     jnp.zeros((1, 1), jnp.float32), loss_ref[...])
    loss_ref[...] = prev + blk


def _dist_argmin_call(x, embedding):
    return pl.pallas_call(
        _dist_argmin_body,
        grid=(N_BLOCKS,),
        in_specs=[
            pl.BlockSpec((BN, DIM), lambda i: (i, 0)),
            pl.BlockSpec((N_EMB, DIM), lambda i: (0, 0)),
            pl.BlockSpec((BN, 1), lambda i: (i, 0)),
            pl.BlockSpec((1, N_EMB), lambda i: (0, 0)),
        ],
        out_specs=[
            pl.BlockSpec((1, 1, N_EMB), lambda i: (i, 0, 0)),
            pl.BlockSpec((1, 1), lambda i: (0, 0)),
        ],
        out_shape=[
            jax.ShapeDtypeStruct((N_BLOCKS, 1, N_EMB), jnp.int32),
            jax.ShapeDtypeStruct((1, 1), jnp.float32),
        ],
    )(x, embedding,
      jnp.sum(x * x, axis=1, keepdims=True),
      jnp.sum(embedding * embedding, axis=1)[None, :])


@functools.cache
def _make_sc_gather():
    info = plsc.get_sparse_core_info()
    nw = info.num_cores * info.num_subcores
    b_per_w = N_ROWS // nw
    mesh = plsc.VectorSubcoreMesh(core_axis_name="c", subcore_axis_name="s")

    @functools.partial(
        pl.kernel,
        mesh=mesh,
        out_type=jax.ShapeDtypeStruct((N_ROWS, DIM), jnp.float32),
        scratch_types=[
            pltpu.VMEM((b_per_w,), jnp.int32),
            pltpu.VMEM((b_per_w, DIM), jnp.float32),
            pltpu.SemaphoreType.DMA,
        ],
        compiler_params=pltpu.CompilerParams(use_tc_tiling_on_sc=False),
    )
    def gather(table_hbm, idx_hbm, out_hbm, idx_v, rows_v, sem):
        wid = lax.axis_index("s") * info.num_cores + lax.axis_index("c")
        base = wid * b_per_w
        pltpu.sync_copy(idx_hbm.at[pl.ds(base, b_per_w)], idx_v)
        pltpu.async_copy(table_hbm.at[idx_v], rows_v, sem).wait()
        pltpu.sync_copy(rows_v, out_hbm.at[pl.ds(base, b_per_w)])

    return gather


def kernel(x, embedding):
    idx_blocks, loss_sum = _dist_argmin_call(x, embedding)
    mapping_inds = idx_blocks.reshape(N_ROWS)
    quantized = _make_sc_gather()(embedding, mapping_inds)
    loss = loss_sum[0, 0] / jnp.float32(N_ROWS * DIM)
    return (quantized, loss, loss, mapping_inds)


# speedup vs baseline: 1.2089x; 1.2089x over previous
import functools

import jax
import jax.numpy as jnp
from jax import lax
from jax.experimental import pallas as pl
from jax.experimental.pallas import tpu as pltpu
from jax.experimental.pallas import tpu_sc as plsc

N_ROWS = 9216
N_EMB = 1024
DIM = 64
BN = 1024
N_BLOCKS = N_ROWS // BN


def _dist_argmin_body(x_ref, emb_ref, xsq_ref, esq_ref, idx_ref, loss_ref):
    x = x_ref[...]
    emb = emb_ref[...]
    x_sq = xsq_ref[...]
    e_sq = esq_ref[...]
    cross2 = lax.dot_general(
        emb, x * jnp.float32(-2.0), (((1,), (1,)), ((), ())),
        preferred_element_type=jnp.float32)
    dis2 = jnp.maximum((e_sq + x_sq) + cross2, 0.0)
    dis = jnp.sqrt(dis2)
    mn = jnp.min(dis, axis=0, keepdims=True)
    iota = lax.broadcasted_iota(jnp.int32, dis.shape, 0)
    idx = jnp.min(jnp.where(dis == mn, iota, N_EMB), axis=0)
    idx_ref[0, 0, :] = idx
    blk = jnp.sum(mn * mn)
    prev = jnp.where(pl.program_id(0) == 0,
                     jnp.zeros((1, 1), jnp.float32), loss_ref[...])
    total = prev + blk
    loss_ref[...] = jnp.where(pl.program_id(0) == N_BLOCKS - 1,
                              total / jnp.float32(N_ROWS * DIM), total)


def _dist_argmin_call(x, embedding):
    return pl.pallas_call(
        _dist_argmin_body,
        grid=(N_BLOCKS,),
        in_specs=[
            pl.BlockSpec((BN, DIM), lambda i: (i, 0)),
            pl.BlockSpec((N_EMB, DIM), lambda i: (0, 0)),
            pl.BlockSpec((1, BN), lambda i: (0, i)),
            pl.BlockSpec((N_EMB, 1), lambda i: (0, 0)),
        ],
        out_specs=[
            pl.BlockSpec((1, 1, BN), lambda i: (i, 0, 0)),
            pl.BlockSpec((1, 1), lambda i: (0, 0)),
        ],
        out_shape=[
            jax.ShapeDtypeStruct((N_BLOCKS, 1, BN), jnp.int32),
            jax.ShapeDtypeStruct((1, 1), jnp.float32),
        ],
    )(x, embedding,
      jnp.sum(x * x, axis=1)[None, :],
      jnp.sum(embedding * embedding, axis=1)[:, None])


@functools.cache
def _make_sc_gather():
    info = plsc.get_sparse_core_info()
    nw = info.num_cores * info.num_subcores
    b_per_w = N_ROWS // nw
    mesh = plsc.VectorSubcoreMesh(core_axis_name="c", subcore_axis_name="s")

    @functools.partial(
        pl.kernel,
        mesh=mesh,
        out_type=jax.ShapeDtypeStruct((N_ROWS, DIM), jnp.float32),
        scratch_types=[
            pltpu.VMEM((b_per_w,), jnp.int32),
            pltpu.VMEM((b_per_w, DIM), jnp.float32),
            pltpu.SemaphoreType.DMA,
        ],
        compiler_params=pltpu.CompilerParams(use_tc_tiling_on_sc=False),
    )
    def gather(table_hbm, idx_hbm, out_hbm, idx_v, rows_v, sem):
        wid = lax.axis_index("s") * info.num_cores + lax.axis_index("c")
        base = wid * b_per_w
        pltpu.sync_copy(idx_hbm.at[pl.ds(base, b_per_w)], idx_v)
        pltpu.async_copy(table_hbm.at[idx_v], rows_v, sem).wait()
        pltpu.sync_copy(rows_v, out_hbm.at[pl.ds(base, b_per_w)])

    return gather


def kernel(x, embedding):
    idx_blocks, loss = _dist_argmin_call(x, embedding)
    mapping_inds = idx_blocks.reshape(N_ROWS)
    quantized = _make_sc_gather()(embedding, mapping_inds)
    loss = loss[0, 0]
    return (quantized, loss, loss, mapping_inds)


# speedup vs baseline: 1.2351x; 1.0217x over previous
import functools

import jax
import jax.numpy as jnp
from jax import lax
from jax.experimental import pallas as pl
from jax.experimental.pallas import tpu as pltpu
from jax.experimental.pallas import tpu_sc as plsc

N_ROWS = 9216
N_EMB = 1024
DIM = 64
BN = 3072
N_BLOCKS = N_ROWS // BN


def _dist_argmin_body(x_ref, emb_ref, xsq_ref, esq_ref, idx_ref, loss_ref):
    x = x_ref[...]
    emb = emb_ref[...]
    x_sq = xsq_ref[...]
    e_sq = esq_ref[...]
    cross2 = lax.dot_general(
        emb, x * jnp.float32(-2.0), (((1,), (1,)), ((), ())),
        preferred_element_type=jnp.float32)
    dis2 = jnp.maximum((e_sq + x_sq) + cross2, 0.0)
    dis = jnp.sqrt(dis2)
    mn = jnp.min(dis, axis=0, keepdims=True)
    iota = lax.broadcasted_iota(jnp.int32, dis.shape, 0)
    idx = jnp.min(jnp.where(dis == mn, iota, N_EMB), axis=0)
    idx_ref[0, 0, :] = idx
    blk = jnp.sum(mn * mn)
    prev = jnp.where(pl.program_id(0) == 0,
                     jnp.zeros((1, 1), jnp.float32), loss_ref[...])
    total = prev + blk
    loss_ref[...] = jnp.where(pl.program_id(0) == N_BLOCKS - 1,
                              total / jnp.float32(N_ROWS * DIM), total)


def _dist_argmin_call(x, embedding):
    return pl.pallas_call(
        _dist_argmin_body,
        grid=(N_BLOCKS,),
        in_specs=[
            pl.BlockSpec((BN, DIM), lambda i: (i, 0)),
            pl.BlockSpec((N_EMB, DIM), lambda i: (0, 0)),
            pl.BlockSpec((1, BN), lambda i: (0, i)),
            pl.BlockSpec((N_EMB, 1), lambda i: (0, 0)),
        ],
        out_specs=[
            pl.BlockSpec((1, 1, BN), lambda i: (i, 0, 0)),
            pl.BlockSpec((1, 1), lambda i: (0, 0)),
        ],
        out_shape=[
            jax.ShapeDtypeStruct((N_BLOCKS, 1, BN), jnp.int32),
            jax.ShapeDtypeStruct((1, 1), jnp.float32),
        ],
    )(x, embedding,
      jnp.sum(x * x, axis=1)[None, :],
      jnp.sum(embedding * embedding, axis=1)[:, None])


@functools.cache
def _make_sc_gather():
    info = plsc.get_sparse_core_info()
    nw = info.num_cores * info.num_subcores
    b_per_w = N_ROWS // nw
    mesh = plsc.VectorSubcoreMesh(core_axis_name="c", subcore_axis_name="s")

    @functools.partial(
        pl.kernel,
        mesh=mesh,
        out_type=jax.ShapeDtypeStruct((N_ROWS, DIM), jnp.float32),
        scratch_types=[
            pltpu.VMEM((b_per_w,), jnp.int32),
            pltpu.VMEM((b_per_w, DIM), jnp.float32),
            pltpu.SemaphoreType.DMA,
        ],
        compiler_params=pltpu.CompilerParams(use_tc_tiling_on_sc=False),
    )
    def gather(table_hbm, idx_hbm, out_hbm, idx_v, rows_v, sem):
        wid = lax.axis_index("s") * info.num_cores + lax.axis_index("c")
        base = wid * b_per_w
        pltpu.sync_copy(idx_hbm.at[pl.ds(base, b_per_w)], idx_v)
        pltpu.async_copy(table_hbm.at[idx_v], rows_v, sem).wait()
        pltpu.sync_copy(rows_v, out_hbm.at[pl.ds(base, b_per_w)])

    return gather


def kernel(x, embedding):
    idx_blocks, loss = _dist_argmin_call(x, embedding)
    mapping_inds = idx_blocks.reshape(N_ROWS)
    quantized = _make_sc_gather()(embedding, mapping_inds)
    loss = loss[0, 0]
    return (quantized, loss, loss, mapping_inds)
